# Initial kernel scaffold; baseline (speedup 1.0000x reference)
import jax
import jax.numpy as jnp
from jax.experimental import pallas as pl


def kernel(x, edge_index, edge_weight, eps, W0, W1, W2, W3, b3, W4, b4):
    raise NotImplementedError("write your pallas kernel here")


# speedup vs baseline: 6.2589x; 6.2589x over previous
import functools

import jax
import jax.numpy as jnp
from jax import lax
from jax.experimental import pallas as pl
from jax.experimental.pallas import tpu as pltpu
from jax.experimental.pallas import tpu_sc as plsc

_CH = 80


def _spmm_sc(h, src1, dst3, w1, zeros, hdim):
    n, gw = h.shape
    nw, nch, _, _ = dst3.shape
    ept = nch * _CH
    nc, ns = 2, 16
    nb = 3
    stride = (n // ns) // 8 * 8
    span = n - stride * (ns - 1)

    mesh = plsc.VectorSubcoreMesh(core_axis_name="c", subcore_axis_name="s")

    scratch = ([pltpu.VMEM((ept,), jnp.int32)]
               + [pltpu.VMEM((1, _CH), jnp.int32)] * nb
               + [pltpu.VMEM((1, _CH), jnp.float32)] * nb
               + [pltpu.VMEM((_CH, gw), jnp.float32)] * nb
               + [pltpu.VMEM_SHARED((n, gw), jnp.float32)]
               + [pltpu.SemaphoreType.DMA] * (4 * nb))

    @functools.partial(
        pl.kernel,
        mesh=mesh,
        out_type=jax.ShapeDtypeStruct((nc, n, gw), jnp.float32),
        scratch_types=scratch,
    )
    def spmm(h_hbm, src_hbm, dst_hbm, w_hbm, z_hbm, out_hbm, src_a, *rest):
        dst_b = rest[0:nb]
        w_b = rest[nb:2 * nb]
        rows_b = rest[2 * nb:3 * nb]
        acc = rest[3 * nb]
        gsem = rest[3 * nb + 1:3 * nb + 1 + nb]
        ssem = rest[3 * nb + 1 + nb:3 * nb + 1 + 2 * nb]
        dsem = rest[3 * nb + 1 + 2 * nb:3 * nb + 1 + 3 * nb]
        wsem = rest[3 * nb + 1 + 3 * nb:3 * nb + 1 + 4 * nb]

        cid = lax.axis_index("c")
        sid = lax.axis_index("s")
        wid = sid * nc + cid
        ebase = wid * ept

        off = sid * stride
        pltpu.sync_copy(z_hbm.at[pl.ds(0, span)], acc.at[pl.ds(off, span)])

        pltpu.sync_copy(src_hbm.at[pl.ds(ebase, ept)], src_a)
        plsc.subcore_barrier()

        def scale(b, w_v):
            rows_v = rows_b[b]
            for g in range(0, _CH, 16):
                w16 = w_v[0, g:g + 16]
                for j in range(16):
                    wb = lax.gather(
                        w16, jnp.full((16, 1), j, jnp.int32),
                        lax.GatherDimensionNumbers(
                            offset_dims=(), collapsed_slice_dims=(0,),
                            start_index_map=(0,)),
                        slice_sizes=(1,),
                        mode=lax.GatherScatterMode.PROMISE_IN_BOUNDS)
                    k = g + j
                    for c0 in range(0, hdim, 16):
                        rows_v[k, c0:c0 + 16] = rows_v[k, c0:c0 + 16] * wb

        def fetch(c, b):
            pltpu.async_copy(dst_hbm.at[wid, c], dst_b[b], dsem[b])
            pltpu.async_copy(w_hbm.at[wid, c], w_b[b], wsem[b])
            pltpu.async_copy(h_hbm.at[src_a.at[pl.ds(c * _CH, _CH)]],
                             rows_b[b], gsem[b])

        def fwait(b):
            pltpu.make_async_copy(dst_hbm.at[wid, 0], dst_b[b], dsem[b]).wait()
            pltpu.make_async_copy(w_hbm.at[wid, 0], w_b[b], wsem[b]).wait()
            pltpu.make_async_copy(h_hbm.at[src_a.at[pl.ds(0, _CH)]],
                                 rows_b[b], gsem[b]).wait()

        def scat(b):
            pltpu.async_copy(rows_b[b], acc.at[dst_b[b].at[0]], ssem[b],
                             add=True)

        def swait(b):
            pltpu.make_async_copy(rows_b[b], acc.at[dst_b[b].at[0]],
                                  ssem[b]).wait()

        last = nch - 1
        for b in range(nb):
            fetch(b, b)

        def triple(i, carry):
            c = nb * i
            for b in range(nb):
                fwait(b)
                scale(b, w_b[b])
                scat(b)
            for b in range(nb):
                swait(b)
                fetch(jnp.minimum(c + nb + b, last), b)
            return carry

        lax.fori_loop(0, (nch - 1) // nb, triple, 0)

        done = (nch - 1) // nb * nb
        for b in range(nch - done):
            fwait(b)
            scale(b, w_b[b])
            scat(b)
        for b in range(nch - done):
            swait(b)
        for b in range(nch - done, nb):
            fwait(b)

        plsc.subcore_barrier()
        pltpu.sync_copy(acc.at[pl.ds(off, span)],
                        out_hbm.at[cid, pl.ds(off, span)])

    return spmm(h, src1, dst3, w1, zeros)


def _mm_body(x_ref, w_ref, o_ref):
    o_ref[...] = jnp.dot(x_ref[...], w_ref[...],
                         preferred_element_type=jnp.float32)


def _tc_matmul(a, b):
    n = a.shape[0]
    m = b.shape[1]
    return pl.pallas_call(
        _mm_body,
        out_shape=jax.ShapeDtypeStruct((n, m), jnp.float32),
    )(a, b)


def _relu_mm_body(p_ref, w_ref, o_ref):
    nn = o_ref.shape[0]
    kk = w_ref.shape[0]
    hid = jnp.maximum(p_ref[0, :nn, :kk] + p_ref[1, :nn, :kk], 0.0)
    o_ref[...] = jnp.dot(hid, w_ref[...], preferred_element_type=jnp.float32)


def _head_body(p_ref, eps_ref, w3_ref, b3_ref, w4_ref, b4_ref,
               zm_ref, zl_ref, z_ref, er_ref):
    nn = eps_ref.shape[0]
    dim = eps_ref.shape[1]
    s = p_ref[0, :nn, :2 * dim] + p_ref[1, :nn, :2 * dim]
    zm = s[:, :dim]
    zl = s[:, dim:]
    z = zm + eps_ref[...] * jnp.exp(zl)
    zm_ref[...] = zm
    zl_ref[...] = zl
    z_ref[...] = z
    h1 = jnp.maximum(
        jnp.dot(z, w3_ref[...], preferred_element_type=jnp.float32)
        + b3_ref[...], 0.0)
    er_ref[...] = jnp.maximum(
        jnp.dot(h1, w4_ref[...], preferred_element_type=jnp.float32)
        + b4_ref[...], 0.0)


def _decode_body(zs_ref, zt_ref, o_ref):
    o_ref[...] = lax.dot_general(
        zs_ref[...], zt_ref[...], (((1,), (1,)), ((), ())),
        preferred_element_type=jnp.float32)


def kernel(x, edge_index, edge_weight, eps, W0, W1, W2, W3, b3, W4, b4):
    n = x.shape[0]
    hdim = W0.shape[1]
    dim = W1.shape[1]
    e = edge_weight.shape[0]
    nw = 32
    ept = e // nw
    nch = ept // _CH
    src1 = edge_index[0]
    dst3 = edge_index[1].reshape(nw, nch, 1, _CH)
    w1 = edge_weight.reshape(nw, nch, 1, _CH)
    span = n - ((n // 16) // 8 * 8) * 15
    zeros = jnp.zeros((span, 128), jnp.float32)

    w0p = jnp.pad(W0, ((0, 0), (0, 128 - hdim)))
    h0 = _tc_matmul(x, w0p)
    p1 = _spmm_sc(h0, src1, dst3, w1, zeros, hdim)

    wc = jnp.pad(jnp.concatenate([W1, W2], axis=1),
                 ((0, 0), (0, 128 - 2 * dim)))
    cmat = pl.pallas_call(
        _relu_mm_body,
        out_shape=jax.ShapeDtypeStruct((n, 128), jnp.float32),
    )(p1, wc)
    p2 = _spmm_sc(cmat, src1, dst3, w1, zeros, 2 * dim)

    z_mean, z_log_std, z, exp_rec = pl.pallas_call(
        _head_body,
        out_shape=[
            jax.ShapeDtypeStruct((n, dim), jnp.float32),
            jax.ShapeDtypeStruct((n, dim), jnp.float32),
            jax.ShapeDtypeStruct((n, dim), jnp.float32),
            jax.ShapeDtypeStruct((n, W4.shape[1]), jnp.float32),
        ],
    )(p2, eps, W3, b3.reshape(1, -1), W4, b4.reshape(1, -1))

    half = dim // 2
    z_src = z[:, :half]
    z_tgt = z[:, half:]
    br = 200
    recon = pl.pallas_call(
        _decode_body,
        grid=(n // br,),
        in_specs=[
            pl.BlockSpec((br, half), lambda i: (i, 0)),
            pl.BlockSpec((n, half), lambda i: (0, 0)),
        ],
        out_specs=pl.BlockSpec((br, n), lambda i: (i, 0)),
        out_shape=jax.ShapeDtypeStruct((n, n), jnp.float32),
    )(z_src, z_tgt)

    return recon.reshape(-1), z_mean, z_log_std, exp_rec


# speedup vs baseline: 6.2662x; 1.0012x over previous
import functools

import jax
import jax.numpy as jnp
from jax import lax
from jax.experimental import pallas as pl
from jax.experimental.pallas import tpu as pltpu
from jax.experimental.pallas import tpu_sc as plsc

_CH = 80


def _spmm_sc(h, src1, dst3, w1, zeros, hdim):
    n, gw = h.shape
    nw, nch, _, _ = dst3.shape
    ept = nch * _CH
    nc, ns = 2, 16
    nb = 3
    stride = (n // ns) // 8 * 8
    span = n - stride * (ns - 1)

    mesh = plsc.VectorSubcoreMesh(core_axis_name="c", subcore_axis_name="s")

    scratch = ([pltpu.VMEM((ept,), jnp.int32)]
               + [pltpu.VMEM((1, _CH), jnp.int32)] * nb
               + [pltpu.VMEM((1, _CH), jnp.float32)] * nb
               + [pltpu.VMEM((_CH, gw), jnp.float32)] * nb
               + [pltpu.VMEM_SHARED((n, gw), jnp.float32)]
               + [pltpu.SemaphoreType.DMA] * (4 * nb))

    @functools.partial(
        pl.kernel,
        mesh=mesh,
        out_type=jax.ShapeDtypeStruct((nc, n, gw), jnp.float32),
        scratch_types=scratch,
    )
    def spmm(h_hbm, src_hbm, dst_hbm, w_hbm, z_hbm, out_hbm, src_a, *rest):
        dst_b = rest[0:nb]
        w_b = rest[nb:2 * nb]
        rows_b = rest[2 * nb:3 * nb]
        acc = rest[3 * nb]
        gsem = rest[3 * nb + 1:3 * nb + 1 + nb]
        ssem = rest[3 * nb + 1 + nb:3 * nb + 1 + 2 * nb]
        dsem = rest[3 * nb + 1 + 2 * nb:3 * nb + 1 + 3 * nb]
        wsem = rest[3 * nb + 1 + 3 * nb:3 * nb + 1 + 4 * nb]

        cid = lax.axis_index("c")
        sid = lax.axis_index("s")
        wid = sid * nc + cid
        ebase = wid * ept

        off = sid * stride
        pltpu.sync_copy(z_hbm.at[pl.ds(0, span)], acc.at[pl.ds(off, span)])

        pltpu.sync_copy(src_hbm.at[pl.ds(ebase, ept)], src_a)
        plsc.subcore_barrier()

        def scale(b, w_v):
            rows_v = rows_b[b]
            for g in range(0, _CH, 16):
                w16 = w_v[0, g:g + 16]
                for j in range(16):
                    wb = lax.gather(
                        w16, jnp.full((16, 1), j, jnp.int32),
                        lax.GatherDimensionNumbers(
                            offset_dims=(), collapsed_slice_dims=(0,),
                            start_index_map=(0,)),
                        slice_sizes=(1,),
                        mode=lax.GatherScatterMode.PROMISE_IN_BOUNDS)
                    k = g + j
                    for c0 in range(0, hdim, 16):
                        rows_v[k, c0:c0 + 16] = rows_v[k, c0:c0 + 16] * wb

        def fetch(c, b):
            pltpu.async_copy(dst_hbm.at[wid, c], dst_b[b], dsem[b])
            pltpu.async_copy(w_hbm.at[wid, c], w_b[b], wsem[b])
            pltpu.async_copy(h_hbm.at[src_a.at[pl.ds(c * _CH, _CH)]],
                             rows_b[b], gsem[b])

        def fwait(b):
            pltpu.make_async_copy(dst_hbm.at[wid, 0], dst_b[b], dsem[b]).wait()
            pltpu.make_async_copy(w_hbm.at[wid, 0], w_b[b], wsem[b]).wait()
            pltpu.make_async_copy(h_hbm.at[src_a.at[pl.ds(0, _CH)]],
                                 rows_b[b], gsem[b]).wait()

        def scat(b):
            pltpu.async_copy(rows_b[b], acc.at[dst_b[b].at[0]], ssem[b],
                             add=True)

        def swait(b):
            pltpu.make_async_copy(rows_b[b], acc.at[dst_b[b].at[0]],
                                  ssem[b]).wait()

        last = nch - 1
        for b in range(nb):
            fetch(b, b)

        def triple(i, carry):
            c = nb * i
            for b in range(nb):
                fwait(b)
                scale(b, w_b[b])
                scat(b)
            for b in range(nb):
                swait(b)
                fetch(jnp.minimum(c + nb + b, last), b)
            return carry

        lax.fori_loop(0, (nch - 1) // nb, triple, 0)

        done = (nch - 1) // nb * nb
        for b in range(nch - done):
            fwait(b)
            scale(b, w_b[b])
            scat(b)
        for b in range(nch - done):
            swait(b)
        for b in range(nch - done, nb):
            fwait(b)

        plsc.subcore_barrier()
        pltpu.sync_copy(acc.at[pl.ds(off, span)],
                        out_hbm.at[cid, pl.ds(off, span)])

    return spmm(h, src1, dst3, w1, zeros)


def _mm_body(x_ref, w_ref, o_ref):
    o_ref[...] = jnp.dot(x_ref[...], w_ref[...],
                         preferred_element_type=jnp.float32)


def _tc_matmul(a, b):
    n = a.shape[0]
    m = b.shape[1]
    return pl.pallas_call(
        _mm_body,
        out_shape=jax.ShapeDtypeStruct((n, m), jnp.float32),
    )(a, b)


def _relu_mm_body(p_ref, w_ref, o_ref):
    nn = o_ref.shape[0]
    kk = w_ref.shape[0]
    hid = jnp.maximum(p_ref[0, :nn, :kk] + p_ref[1, :nn, :kk], 0.0)
    o_ref[...] = jnp.dot(hid, w_ref[...], preferred_element_type=jnp.float32)


def _head_body(p_ref, eps_ref, w3_ref, b3_ref, w4_ref, b4_ref,
               zm_ref, zl_ref, z_ref, er_ref):
    nn = eps_ref.shape[0]
    dim = eps_ref.shape[1]
    s = p_ref[0, :nn, :2 * dim] + p_ref[1, :nn, :2 * dim]
    zm = s[:, :dim]
    zl = s[:, dim:]
    z = zm + eps_ref[...] * jnp.exp(zl)
    zm_ref[...] = zm
    zl_ref[...] = zl
    z_ref[...] = z
    h1 = jnp.maximum(
        jnp.dot(z, w3_ref[...], preferred_element_type=jnp.float32)
        + b3_ref[...], 0.0)
    er_ref[...] = jnp.maximum(
        jnp.dot(h1, w4_ref[...], preferred_element_type=jnp.float32)
        + b4_ref[...], 0.0)


def _decode_body(zs_ref, zt_ref, o_ref):
    o_ref[...] = lax.dot_general(
        zs_ref[...], zt_ref[...], (((1,), (1,)), ((), ())),
        preferred_element_type=jnp.float32)


def kernel(x, edge_index, edge_weight, eps, W0, W1, W2, W3, b3, W4, b4):
    n = x.shape[0]
    hdim = W0.shape[1]
    dim = W1.shape[1]
    e = edge_weight.shape[0]
    nw = 32
    ept = e // nw
    nch = ept // _CH
    src1 = edge_index[0]
    dst3 = edge_index[1].reshape(nw, nch, 1, _CH)
    w1 = edge_weight.reshape(nw, nch, 1, _CH)
    span = n - ((n // 16) // 8 * 8) * 15
    zeros = jnp.zeros((span, 128), jnp.float32)

    w0p = jnp.pad(W0, ((0, 0), (0, 128 - hdim)))
    h0 = _tc_matmul(x, w0p)
    p1 = _spmm_sc(h0, src1, dst3, w1, zeros, hdim)

    wc = jnp.pad(jnp.concatenate([W1, W2], axis=1),
                 ((0, 0), (0, 128 - 2 * dim)))
    cmat = pl.pallas_call(
        _relu_mm_body,
        out_shape=jax.ShapeDtypeStruct((n, 128), jnp.float32),
    )(p1, wc)
    p2 = _spmm_sc(cmat, src1, dst3, w1, zeros, 2 * dim)

    z_mean, z_log_std, z, exp_rec = pl.pallas_call(
        _head_body,
        out_shape=[
            jax.ShapeDtypeStruct((n, dim), jnp.float32),
            jax.ShapeDtypeStruct((n, dim), jnp.float32),
            jax.ShapeDtypeStruct((n, dim), jnp.float32),
            jax.ShapeDtypeStruct((n, W4.shape[1]), jnp.float32),
        ],
    )(p2, eps, W3, b3.reshape(1, -1), W4, b4.reshape(1, -1))

    half = dim // 2
    z_src = z[:, :half]
    z_tgt = z[:, half:]
    br = 400
    recon = pl.pallas_call(
        _decode_body,
        grid=(n // br,),
        in_specs=[
            pl.BlockSpec((br, half), lambda i: (i, 0)),
            pl.BlockSpec((n, half), lambda i: (0, 0)),
        ],
        out_specs=pl.BlockSpec((br, n), lambda i: (i, 0)),
        out_shape=jax.ShapeDtypeStruct((n, n), jnp.float32),
    )(z_src, z_tgt)

    return recon.reshape(-1), z_mean, z_log_std, exp_rec
